# baseline (speedup 1.0000x reference)
import functools
import math

import jax
import jax.numpy as jnp
from jax import lax
from jax.experimental import pallas as pl
from jax.experimental.pallas import tpu as pltpu
from jax.experimental.pallas import tpu_sc as plsc

_NUM_CORES = 2
_NUM_TILES = 16
_CHUNK = 128


def _make_sc_kernels(n_pad, ch, hid):
    rows_per_tile = n_pad // _NUM_TILES
    ch_per_tile = ch // (_NUM_TILES * _NUM_CORES)
    cpt_pad = ch_per_tile + 8
    nbuf = 3
    mesh = plsc.VectorSubcoreMesh(core_axis_name="c", subcore_axis_name="s")

    nidx = nbuf + 1
    period = (nbuf * nidx) // math.gcd(nbuf, nidx)

    @functools.partial(
        pl.kernel,
        out_type=jax.ShapeDtypeStruct((_NUM_CORES, n_pad, hid), jnp.float32),
        mesh=mesh,
        scratch_types=(
            [pltpu.VMEM((nidx, 2, _CHUNK), jnp.int32),
             pltpu.VMEM((nbuf, _CHUNK, hid), jnp.float32),
             pltpu.VMEM_SHARED((n_pad, hid), jnp.float32)]
            + [pltpu.SemaphoreType.DMA] * (nbuf + nidx)
        ),
    )
    def spmm(hh_hbm, sd_hbm, zeros_hbm, out_hbm, *rest):
        sdv3, rows3, aggs = rest[0], rest[1], rest[2]
        sdv = [sdv3.at[j] for j in range(nidx)]
        rows = [rows3.at[b] for b in range(nbuf)]
        sems = list(rest[3:3 + nbuf])
        isems = list(rest[3 + nbuf:])
        cid = lax.axis_index("c")
        sid = lax.axis_index("s")
        sl = pl.ds(sid * rows_per_tile, rows_per_tile)
        base = (cid * _NUM_TILES + sid) * ch_per_tile
        pltpu.sync_copy(zeros_hbm, aggs.at[sl])
        plsc.subcore_barrier()

        def fire_idx(c, j):
            pltpu.async_copy(sd_hbm.at[base + c], sdv[j], isems[j])

        def fire_gather(j, b):
            pltpu.make_async_copy(sd_hbm.at[base], sdv[j], isems[j]).wait()
            pltpu.async_copy(hh_hbm.at[sdv[j].at[0]], rows[b], sems[b])

        def drain_scatter(j, b, scatter=True):
            pltpu.make_async_copy(hh_hbm.at[sdv[j].at[0]], rows[b],
                                  sems[b]).wait()
            if scatter:
                pltpu.sync_copy(rows[b], aggs.at[sdv[j].at[1]], add=True)

        for c in range(nbuf):
            fire_idx(c, c % nidx)
        for c in range(nbuf - 1):
            fire_gather(c % nidx, c % nbuf)

        def step(c, k):
            fire_gather((k + nbuf - 1) % nidx, (k + nbuf - 1) % nbuf)
            drain_scatter(k % nidx, k % nbuf)
            fire_idx(c + nbuf, (k + nbuf) % nidx)

        def body(tt, carry):
            for k in range(period):
                step(period * tt + k, k)
            return carry

        main = ch_per_tile // period
        lax.fori_loop(0, main, body, 0)
        for i in range(ch_per_tile - main * period):
            c = main * period + i
            step(c, i)
        for c in range(ch_per_tile, ch_per_tile + nbuf - 1):
            drain_scatter(c % nidx, c % nbuf, scatter=False)
        j = (ch_per_tile + nbuf - 1) % nidx
        pltpu.make_async_copy(sd_hbm.at[base], sdv[j], isems[j]).wait()

        plsc.subcore_barrier()
        pltpu.sync_copy(aggs.at[sl], out_hbm.at[cid, sl])

    @functools.partial(
        pl.kernel,
        out_type=jax.ShapeDtypeStruct((_NUM_CORES, n_pad, hid), jnp.float32),
        mesh=mesh,
        scratch_types=[
            pltpu.VMEM((cpt_pad, _CHUNK), jnp.int32),
            pltpu.VMEM((_CHUNK, hid), jnp.float32),
            pltpu.VMEM_SHARED((n_pad, hid), jnp.float32),
        ],
    )
    def degree(dst_hbm, ones_hbm, zeros_hbm, out_hbm, dsti, ones_v, degs):
        cid = lax.axis_index("c")
        sid = lax.axis_index("s")
        sl = pl.ds(sid * rows_per_tile, rows_per_tile)
        base = (cid * _NUM_TILES + sid) * ch_per_tile
        pltpu.sync_copy(dst_hbm.at[pl.ds(base, cpt_pad)], dsti)
        pltpu.sync_copy(zeros_hbm, degs.at[sl])
        pltpu.sync_copy(ones_hbm, ones_v)
        plsc.subcore_barrier()

        def body(t, carry):
            pltpu.sync_copy(ones_v, degs.at[dsti.at[t]], add=True)
            return carry

        lax.fori_loop(0, ch_per_tile, body, 0)
        plsc.subcore_barrier()
        pltpu.sync_copy(degs.at[sl], out_hbm.at[cid, sl])

    return spmm, degree


def _prep_body(n, feat_ref, degp_ref, dinv_ref, hh_ref, pooled_ref):
    feat = feat_ref[...]
    deg = degp_ref[0, :, 0:1] + degp_ref[1, :, 0:1]
    dinv = lax.rsqrt(jnp.maximum(deg, 1.0))
    dinv_ref[...] = dinv
    hh_ref[...] = feat * dinv
    pooled_ref[...] = jnp.sum(feat, axis=0, keepdims=True)


def _layer_body(n, h_ref, agg_ref, dinv_ref, w_ref, b_ref, g_ref, beta_ref,
                hout_ref, hh_ref, pooled_ref):
    h = h_ref[...]
    n_pad, hid = h.shape
    dinv = dinv_ref[...]
    w = w_ref[...]
    x1 = -((agg_ref[0] + agg_ref[1]) * dinv)
    z = (jnp.dot(h, w[:hid], preferred_element_type=jnp.float32)
         + jnp.dot(x1, w[hid:], preferred_element_type=jnp.float32)
         + b_ref[...])
    rowmask = lax.broadcasted_iota(jnp.int32, (n_pad, 1), 0) < n
    zm = jnp.where(rowmask, z, 0.0)
    mean = jnp.sum(zm, axis=0, keepdims=True) / n
    d = jnp.where(rowmask, z - mean, 0.0)
    var = jnp.sum(d * d, axis=0, keepdims=True) / n
    hn = (z - mean) * lax.rsqrt(var + 1e-5) * g_ref[...] + beta_ref[...]
    hn = jnp.where(rowmask, jnp.maximum(hn, 0.0), 0.0)
    hout_ref[...] = hn
    pooled_ref[...] = jnp.sum(hn, axis=0, keepdims=True)
    hh_ref[...] = hn * dinv


def _head_body(p_ref, w_ref, b_ref, lp_ref, pm_ref):
    p = p_ref[...]
    w = w_ref[...]
    b = b_ref[...]
    reps = p.shape[0]
    s = jnp.zeros((1, w.shape[2]), jnp.float32)
    for i in range(reps):
        s = s + jnp.dot(p[i:i + 1], w[i], preferred_element_type=jnp.float32)
        s = s + b[i:i + 1]
    m = jnp.max(s)
    lse = m + jnp.log(jnp.sum(jnp.exp(s - m)))
    lp_ref[...] = s - lse
    pm_ref[...] = jnp.mean(p[1:], axis=0, keepdims=True)


def kernel(feat, edge_index, cheb_W, cheb_b, bn_gamma, bn_beta, lp_W, lp_b):
    n, in_dim = feat.shape
    e = edge_index.shape[1]
    num_layers, two_hid, hid = cheb_W.shape
    half = hid // 2
    out_dim = lp_W.shape[2]

    align = _NUM_TILES * 8
    n_pad = ((n + align - 1) // align) * align
    if n_pad == n:
        n_pad += align
    nch = (e + _CHUNK - 1) // _CHUNK
    grp = _NUM_TILES * _NUM_CORES * 8
    ch = ((nch + grp - 1) // grp) * grp
    pad_e = (ch + 8) * _CHUNK - e

    src = edge_index[0]
    dst = edge_index[1]
    lane = jnp.arange(pad_e, dtype=jnp.int32) % (n_pad - n)
    src_p = jnp.concatenate([src, n + lane]).reshape(ch + 8, _CHUNK)
    dst_p = jnp.concatenate([dst, n + lane]).reshape(ch + 8, _CHUNK)
    sd_p = jnp.stack([src_p, dst_p], axis=1)

    spmm, degree = _make_sc_kernels(n_pad, ch, hid)
    rows_per_tile = n_pad // _NUM_TILES
    zeros_agg = jnp.zeros((rows_per_tile, hid), jnp.float32)
    zeros_deg = zeros_agg
    ones_deg = jnp.ones((_CHUNK, hid), jnp.float32)

    deg_part = degree(dst_p, ones_deg, zeros_deg)

    feat_pad = jnp.zeros((n_pad, in_dim), jnp.float32).at[:n].set(feat)
    dinv, hh, pooled0 = pl.pallas_call(
        functools.partial(_prep_body, n),
        out_shape=(
            jax.ShapeDtypeStruct((n_pad, 1), jnp.float32),
            jax.ShapeDtypeStruct((n_pad, hid), jnp.float32),
            jax.ShapeDtypeStruct((1, hid), jnp.float32),
        ),
    )(feat_pad, deg_part)

    layer_call = pl.pallas_call(
        functools.partial(_layer_body, n),
        out_shape=(
            jax.ShapeDtypeStruct((n_pad, hid), jnp.float32),
            jax.ShapeDtypeStruct((n_pad, hid), jnp.float32),
            jax.ShapeDtypeStruct((1, hid), jnp.float32),
        ),
    )

    h = feat_pad
    pooled = [pooled0]
    for i in range(num_layers):
        agg = spmm(hh, sd_p, zeros_agg)
        h, hh, p = layer_call(
            h, agg, dinv, cheb_W[i], cheb_b[i].reshape(1, hid),
            bn_gamma[i].reshape(1, hid), bn_beta[i].reshape(1, hid))
        pooled.append(p)

    p_all = jnp.concatenate(pooled, axis=0)
    log_probs, pooled_mean = pl.pallas_call(
        _head_body,
        out_shape=(
            jax.ShapeDtypeStruct((1, out_dim), jnp.float32),
            jax.ShapeDtypeStruct((1, hid), jnp.float32),
        ),
    )(p_all, lp_W, lp_b)
    return log_probs, pooled_mean


# speedup vs baseline: 1.1460x; 1.1460x over previous
import functools
import math

import jax
import jax.numpy as jnp
from jax import lax
from jax.experimental import pallas as pl
from jax.experimental.pallas import tpu as pltpu
from jax.experimental.pallas import tpu_sc as plsc

_NUM_CORES = 2
_NUM_TILES = 16
_CHUNK = 128


def _make_sc_kernels(n_pad, ch, hid):
    rows_per_tile = n_pad // _NUM_TILES
    ch_per_tile = ch // (_NUM_TILES * _NUM_CORES)
    cpt_pad = ch_per_tile + 8
    nbuf = 3
    mesh = plsc.VectorSubcoreMesh(core_axis_name="c", subcore_axis_name="s")

    nidx = nbuf + 1
    period = (nbuf * nidx) // math.gcd(nbuf, nidx)

    @functools.partial(
        pl.kernel,
        out_type=jax.ShapeDtypeStruct((_NUM_CORES, n_pad, hid), jnp.float32),
        mesh=mesh,
        scratch_types=(
            [pltpu.VMEM((nidx, 2, _CHUNK), jnp.int32),
             pltpu.VMEM((nbuf, _CHUNK, hid), jnp.float32),
             pltpu.VMEM_SHARED((n_pad, hid), jnp.float32)]
            + [pltpu.SemaphoreType.DMA] * (2 * nbuf + nidx)
        ),
    )
    def spmm(hh_hbm, sd_hbm, zeros_hbm, out_hbm, *rest):
        sdv3, rows3, aggs = rest[0], rest[1], rest[2]
        sdv = [sdv3.at[j] for j in range(nidx)]
        rows = [rows3.at[b] for b in range(nbuf)]
        sems = list(rest[3:3 + nbuf])
        ssems = list(rest[3 + nbuf:3 + 2 * nbuf])
        isems = list(rest[3 + 2 * nbuf:])
        cid = lax.axis_index("c")
        sid = lax.axis_index("s")
        sl = pl.ds(sid * rows_per_tile, rows_per_tile)
        base = (cid * _NUM_TILES + sid) * ch_per_tile
        pltpu.sync_copy(zeros_hbm, aggs.at[sl])
        plsc.subcore_barrier()

        def fire_idx(c, j):
            pltpu.async_copy(sd_hbm.at[base + c], sdv[j], isems[j])

        def fire_gather(j, b):
            pltpu.make_async_copy(sd_hbm.at[base], sdv[j], isems[j]).wait()
            pltpu.async_copy(hh_hbm.at[sdv[j].at[0]], rows[b], sems[b])

        def fire_scatter(j, b):
            pltpu.make_async_copy(hh_hbm.at[sdv[j].at[0]], rows[b],
                                  sems[b]).wait()
            pltpu.async_copy(rows[b], aggs.at[sdv[j].at[1]], ssems[b],
                             add=True)

        def wait_scatter(j, b):
            pltpu.make_async_copy(rows[b], aggs.at[sdv[j].at[1]],
                                  ssems[b]).wait()

        def drain_gather(j, b):
            pltpu.make_async_copy(hh_hbm.at[sdv[j].at[0]], rows[b],
                                  sems[b]).wait()

        for c in range(nbuf):
            fire_idx(c, c % nidx)
        for c in range(nbuf - 1):
            fire_gather(c % nidx, c % nbuf)

        def step(c, k, first=False):
            fire_scatter(k % nidx, k % nbuf)
            if not first:
                wait_scatter((k - 1) % nidx, (k + nbuf - 1) % nbuf)
            fire_gather((k + nbuf - 1) % nidx, (k + nbuf - 1) % nbuf)
            fire_idx(c + nbuf, (k + nbuf) % nidx)

        def body(tt, carry):
            for k in range(period):
                step(period * tt + k, k)
            return carry

        main = ch_per_tile // period
        for k in range(period):
            step(k, k, first=(k == 0))
        lax.fori_loop(1, main, body, 0)
        for i in range(ch_per_tile - main * period):
            c = main * period + i
            step(c, i)
        lk = ch_per_tile - 1
        wait_scatter(lk % nidx, lk % nbuf)
        for c in range(ch_per_tile, ch_per_tile + nbuf - 1):
            drain_gather(c % nidx, c % nbuf)
        j = (ch_per_tile + nbuf - 1) % nidx
        pltpu.make_async_copy(sd_hbm.at[base], sdv[j], isems[j]).wait()

        plsc.subcore_barrier()
        pltpu.sync_copy(aggs.at[sl], out_hbm.at[cid, sl])

    @functools.partial(
        pl.kernel,
        out_type=jax.ShapeDtypeStruct((_NUM_CORES, n_pad, hid), jnp.float32),
        mesh=mesh,
        scratch_types=[
            pltpu.VMEM((cpt_pad, _CHUNK), jnp.int32),
            pltpu.VMEM((_CHUNK, hid), jnp.float32),
            pltpu.VMEM_SHARED((n_pad, hid), jnp.float32),
        ],
    )
    def degree(dst_hbm, ones_hbm, zeros_hbm, out_hbm, dsti, ones_v, degs):
        cid = lax.axis_index("c")
        sid = lax.axis_index("s")
        sl = pl.ds(sid * rows_per_tile, rows_per_tile)
        base = (cid * _NUM_TILES + sid) * ch_per_tile
        pltpu.sync_copy(dst_hbm.at[pl.ds(base, cpt_pad)], dsti)
        pltpu.sync_copy(zeros_hbm, degs.at[sl])
        pltpu.sync_copy(ones_hbm, ones_v)
        plsc.subcore_barrier()

        def body(t, carry):
            pltpu.sync_copy(ones_v, degs.at[dsti.at[t]], add=True)
            return carry

        lax.fori_loop(0, ch_per_tile, body, 0)
        plsc.subcore_barrier()
        pltpu.sync_copy(degs.at[sl], out_hbm.at[cid, sl])

    return spmm, degree


def _prep_body(n, feat_ref, degp_ref, dinv_ref, hh_ref, pooled_ref):
    feat = feat_ref[...]
    deg = degp_ref[0, :, 0:1] + degp_ref[1, :, 0:1]
    dinv = lax.rsqrt(jnp.maximum(deg, 1.0))
    dinv_ref[...] = dinv
    hh_ref[...] = feat * dinv
    pooled_ref[...] = jnp.sum(feat, axis=0, keepdims=True)


def _layer_body(n, h_ref, agg_ref, dinv_ref, w_ref, b_ref, g_ref, beta_ref,
                hout_ref, hh_ref, pooled_ref):
    h = h_ref[...]
    n_pad, hid = h.shape
    dinv = dinv_ref[...]
    w = w_ref[...]
    x1 = -((agg_ref[0] + agg_ref[1]) * dinv)
    z = (jnp.dot(h, w[:hid], preferred_element_type=jnp.float32)
         + jnp.dot(x1, w[hid:], preferred_element_type=jnp.float32)
         + b_ref[...])
    rowmask = lax.broadcasted_iota(jnp.int32, (n_pad, 1), 0) < n
    zm = jnp.where(rowmask, z, 0.0)
    mean = jnp.sum(zm, axis=0, keepdims=True) / n
    d = jnp.where(rowmask, z - mean, 0.0)
    var = jnp.sum(d * d, axis=0, keepdims=True) / n
    hn = (z - mean) * lax.rsqrt(var + 1e-5) * g_ref[...] + beta_ref[...]
    hn = jnp.where(rowmask, jnp.maximum(hn, 0.0), 0.0)
    hout_ref[...] = hn
    pooled_ref[...] = jnp.sum(hn, axis=0, keepdims=True)
    hh_ref[...] = hn * dinv


def _head_body(p_ref, w_ref, b_ref, lp_ref, pm_ref):
    p = p_ref[...]
    w = w_ref[...]
    b = b_ref[...]
    reps = p.shape[0]
    s = jnp.zeros((1, w.shape[2]), jnp.float32)
    for i in range(reps):
        s = s + jnp.dot(p[i:i + 1], w[i], preferred_element_type=jnp.float32)
        s = s + b[i:i + 1]
    m = jnp.max(s)
    lse = m + jnp.log(jnp.sum(jnp.exp(s - m)))
    lp_ref[...] = s - lse
    pm_ref[...] = jnp.mean(p[1:], axis=0, keepdims=True)


def kernel(feat, edge_index, cheb_W, cheb_b, bn_gamma, bn_beta, lp_W, lp_b):
    n, in_dim = feat.shape
    e = edge_index.shape[1]
    num_layers, two_hid, hid = cheb_W.shape
    half = hid // 2
    out_dim = lp_W.shape[2]

    align = _NUM_TILES * 8
    n_pad = ((n + align - 1) // align) * align
    if n_pad == n:
        n_pad += align
    nch = (e + _CHUNK - 1) // _CHUNK
    grp = _NUM_TILES * _NUM_CORES * 8
    ch = ((nch + grp - 1) // grp) * grp
    pad_e = (ch + 8) * _CHUNK - e

    src = edge_index[0]
    dst = edge_index[1]
    lane = jnp.arange(pad_e, dtype=jnp.int32) % (n_pad - n)
    src_p = jnp.concatenate([src, n + lane]).reshape(ch + 8, _CHUNK)
    dst_p = jnp.concatenate([dst, n + lane]).reshape(ch + 8, _CHUNK)
    sd_p = jnp.stack([src_p, dst_p], axis=1)

    spmm, degree = _make_sc_kernels(n_pad, ch, hid)
    rows_per_tile = n_pad // _NUM_TILES
    zeros_agg = jnp.zeros((rows_per_tile, hid), jnp.float32)
    zeros_deg = zeros_agg
    ones_deg = jnp.ones((_CHUNK, hid), jnp.float32)

    deg_part = degree(dst_p, ones_deg, zeros_deg)

    feat_pad = jnp.zeros((n_pad, in_dim), jnp.float32).at[:n].set(feat)
    dinv, hh, pooled0 = pl.pallas_call(
        functools.partial(_prep_body, n),
        out_shape=(
            jax.ShapeDtypeStruct((n_pad, 1), jnp.float32),
            jax.ShapeDtypeStruct((n_pad, hid), jnp.float32),
            jax.ShapeDtypeStruct((1, hid), jnp.float32),
        ),
    )(feat_pad, deg_part)

    layer_call = pl.pallas_call(
        functools.partial(_layer_body, n),
        out_shape=(
            jax.ShapeDtypeStruct((n_pad, hid), jnp.float32),
            jax.ShapeDtypeStruct((n_pad, hid), jnp.float32),
            jax.ShapeDtypeStruct((1, hid), jnp.float32),
        ),
    )

    h = feat_pad
    pooled = [pooled0]
    for i in range(num_layers):
        agg = spmm(hh, sd_p, zeros_agg)
        h, hh, p = layer_call(
            h, agg, dinv, cheb_W[i], cheb_b[i].reshape(1, hid),
            bn_gamma[i].reshape(1, hid), bn_beta[i].reshape(1, hid))
        pooled.append(p)

    p_all = jnp.concatenate(pooled, axis=0)
    log_probs, pooled_mean = pl.pallas_call(
        _head_body,
        out_shape=(
            jax.ShapeDtypeStruct((1, out_dim), jnp.float32),
            jax.ShapeDtypeStruct((1, hid), jnp.float32),
        ),
    )(p_all, lp_W, lp_b)
    return log_probs, pooled_mean


# speedup vs baseline: 1.1475x; 1.0013x over previous
import functools
import math

import jax
import jax.numpy as jnp
from jax import lax
from jax.experimental import pallas as pl
from jax.experimental.pallas import tpu as pltpu
from jax.experimental.pallas import tpu_sc as plsc

_NUM_CORES = 2
_NUM_TILES = 16
_CHUNK = 128


def _make_sc_kernels(n_pad, ch, hid):
    rows_per_tile = n_pad // _NUM_TILES
    ch_per_tile = ch // (_NUM_TILES * _NUM_CORES)
    cpt_pad = ch_per_tile + 8
    nbuf = 3
    mesh = plsc.VectorSubcoreMesh(core_axis_name="c", subcore_axis_name="s")

    nidx = nbuf + 1
    period = (nbuf * nidx) // math.gcd(nbuf, nidx)

    @functools.partial(
        pl.kernel,
        out_type=jax.ShapeDtypeStruct((_NUM_CORES, n_pad, hid), jnp.float32),
        mesh=mesh,
        scratch_types=(
            [pltpu.VMEM((nidx, 2, _CHUNK), jnp.int32),
             pltpu.VMEM((nbuf, _CHUNK, hid), jnp.float32),
             pltpu.VMEM_SHARED((n_pad, hid), jnp.float32)]
            + [pltpu.SemaphoreType.DMA] * (2 * nbuf + nidx)
        ),
    )
    def spmm(hh_hbm, sd_hbm, zeros_hbm, out_hbm, *rest):
        sdv3, rows3, aggs = rest[0], rest[1], rest[2]
        sdv = [sdv3.at[j] for j in range(nidx)]
        rows = [rows3.at[b] for b in range(nbuf)]
        sems = list(rest[3:3 + nbuf])
        ssems = list(rest[3 + nbuf:3 + 2 * nbuf])
        isems = list(rest[3 + 2 * nbuf:])
        cid = lax.axis_index("c")
        sid = lax.axis_index("s")
        sl = pl.ds(sid * rows_per_tile, rows_per_tile)
        base = (cid * _NUM_TILES + sid) * ch_per_tile
        pltpu.sync_copy(zeros_hbm, aggs.at[sl])
        plsc.subcore_barrier()

        def fire_idx(c, j):
            pltpu.async_copy(sd_hbm.at[base + c], sdv[j], isems[j])

        def fire_gather(j, b):
            pltpu.make_async_copy(sd_hbm.at[base], sdv[j], isems[j]).wait()
            pltpu.async_copy(hh_hbm.at[sdv[j].at[0]], rows[b], sems[b])

        def fire_scatter(j, b):
            pltpu.make_async_copy(hh_hbm.at[sdv[j].at[0]], rows[b],
                                  sems[b]).wait()
            pltpu.async_copy(rows[b], aggs.at[sdv[j].at[1]], ssems[b],
                             add=True)

        def wait_scatter(j, b):
            pltpu.make_async_copy(rows[b], aggs.at[sdv[j].at[1]],
                                  ssems[b]).wait()

        def drain_gather(j, b):
            pltpu.make_async_copy(hh_hbm.at[sdv[j].at[0]], rows[b],
                                  sems[b]).wait()

        for c in range(nbuf):
            fire_idx(c, c % nidx)
        for c in range(nbuf - 1):
            fire_gather(c % nidx, c % nbuf)

        def step(c, k, first=False):
            fire_scatter(k % nidx, k % nbuf)
            if not first:
                wait_scatter((k - 1) % nidx, (k + nbuf - 1) % nbuf)
            fire_gather((k + nbuf - 1) % nidx, (k + nbuf - 1) % nbuf)
            fire_idx(c + nbuf, (k + nbuf) % nidx)

        def body(tt, carry):
            for k in range(period):
                step(period * tt + k, k)
            return carry

        main = ch_per_tile // period
        for k in range(period):
            step(k, k, first=(k == 0))
        lax.fori_loop(1, main, body, 0)
        for i in range(ch_per_tile - main * period):
            c = main * period + i
            step(c, i)
        lk = ch_per_tile - 1
        wait_scatter(lk % nidx, lk % nbuf)
        for c in range(ch_per_tile, ch_per_tile + nbuf - 1):
            drain_gather(c % nidx, c % nbuf)
        j = (ch_per_tile + nbuf - 1) % nidx
        pltpu.make_async_copy(sd_hbm.at[base], sdv[j], isems[j]).wait()

        plsc.subcore_barrier()
        pltpu.sync_copy(aggs.at[sl], out_hbm.at[cid, sl])

    ndeg = 4

    @functools.partial(
        pl.kernel,
        out_type=jax.ShapeDtypeStruct((_NUM_CORES, n_pad, hid), jnp.float32),
        mesh=mesh,
        scratch_types=(
            [pltpu.VMEM((cpt_pad, _CHUNK), jnp.int32),
             pltpu.VMEM((_CHUNK, hid), jnp.float32),
             pltpu.VMEM_SHARED((n_pad, hid), jnp.float32)]
            + [pltpu.SemaphoreType.DMA] * ndeg
        ),
    )
    def degree(dst_hbm, ones_hbm, zeros_hbm, out_hbm, dsti, ones_v, degs,
               *dsems):
        cid = lax.axis_index("c")
        sid = lax.axis_index("s")
        sl = pl.ds(sid * rows_per_tile, rows_per_tile)
        base = (cid * _NUM_TILES + sid) * ch_per_tile
        pltpu.sync_copy(dst_hbm.at[pl.ds(base, cpt_pad)], dsti)
        pltpu.sync_copy(zeros_hbm, degs.at[sl])
        pltpu.sync_copy(ones_hbm, ones_v)
        plsc.subcore_barrier()

        def fire(t, k):
            pltpu.async_copy(ones_v, degs.at[dsti.at[t]], dsems[k],
                             add=True)

        def wait(t, k):
            pltpu.make_async_copy(ones_v, degs.at[dsti.at[t]],
                                  dsems[k]).wait()

        for k in range(ndeg):
            fire(k, k)

        def body(tt, carry):
            for k in range(ndeg):
                t = ndeg * tt + k
                wait(t - ndeg, k)
                fire(t, k)
            return carry

        lax.fori_loop(1, ch_per_tile // ndeg, body, 0)
        for k in range(ndeg):
            wait(ch_per_tile - ndeg + k, k)
        plsc.subcore_barrier()
        pltpu.sync_copy(degs.at[sl], out_hbm.at[cid, sl])

    return spmm, degree


def _prep_body(n, feat_ref, degp_ref, dinv_ref, hh_ref, pooled_ref):
    feat = feat_ref[...]
    deg = degp_ref[0, :, 0:1] + degp_ref[1, :, 0:1]
    dinv = lax.rsqrt(jnp.maximum(deg, 1.0))
    dinv_ref[...] = dinv
    hh_ref[...] = feat * dinv
    pooled_ref[...] = jnp.sum(feat, axis=0, keepdims=True)


def _layer_body(n, h_ref, agg_ref, dinv_ref, w_ref, b_ref, g_ref, beta_ref,
                hout_ref, hh_ref, pooled_ref):
    h = h_ref[...]
    n_pad, hid = h.shape
    dinv = dinv_ref[...]
    w = w_ref[...]
    x1 = -((agg_ref[0] + agg_ref[1]) * dinv)
    z = (jnp.dot(h, w[:hid], preferred_element_type=jnp.float32)
         + jnp.dot(x1, w[hid:], preferred_element_type=jnp.float32)
         + b_ref[...])
    rowmask = lax.broadcasted_iota(jnp.int32, (n_pad, 1), 0) < n
    zm = jnp.where(rowmask, z, 0.0)
    mean = jnp.sum(zm, axis=0, keepdims=True) / n
    d = jnp.where(rowmask, z - mean, 0.0)
    var = jnp.sum(d * d, axis=0, keepdims=True) / n
    hn = (z - mean) * lax.rsqrt(var + 1e-5) * g_ref[...] + beta_ref[...]
    hn = jnp.where(rowmask, jnp.maximum(hn, 0.0), 0.0)
    hout_ref[...] = hn
    pooled_ref[...] = jnp.sum(hn, axis=0, keepdims=True)
    hh_ref[...] = hn * dinv


def _head_body(p_ref, w_ref, b_ref, lp_ref, pm_ref):
    p = p_ref[...]
    w = w_ref[...]
    b = b_ref[...]
    reps = p.shape[0]
    s = jnp.zeros((1, w.shape[2]), jnp.float32)
    for i in range(reps):
        s = s + jnp.dot(p[i:i + 1], w[i], preferred_element_type=jnp.float32)
        s = s + b[i:i + 1]
    m = jnp.max(s)
    lse = m + jnp.log(jnp.sum(jnp.exp(s - m)))
    lp_ref[...] = s - lse
    pm_ref[...] = jnp.mean(p[1:], axis=0, keepdims=True)


def kernel(feat, edge_index, cheb_W, cheb_b, bn_gamma, bn_beta, lp_W, lp_b):
    n, in_dim = feat.shape
    e = edge_index.shape[1]
    num_layers, two_hid, hid = cheb_W.shape
    half = hid // 2
    out_dim = lp_W.shape[2]

    align = _NUM_TILES * 8
    n_pad = ((n + align - 1) // align) * align
    if n_pad == n:
        n_pad += align
    nch = (e + _CHUNK - 1) // _CHUNK
    grp = _NUM_TILES * _NUM_CORES * 8
    ch = ((nch + grp - 1) // grp) * grp
    pad_e = (ch + 8) * _CHUNK - e

    src = edge_index[0]
    dst = edge_index[1]
    lane = jnp.arange(pad_e, dtype=jnp.int32) % (n_pad - n)
    src_p = jnp.concatenate([src, n + lane]).reshape(ch + 8, _CHUNK)
    dst_p = jnp.concatenate([dst, n + lane]).reshape(ch + 8, _CHUNK)
    sd_p = jnp.stack([src_p, dst_p], axis=1)

    spmm, degree = _make_sc_kernels(n_pad, ch, hid)
    rows_per_tile = n_pad // _NUM_TILES
    zeros_agg = jnp.zeros((rows_per_tile, hid), jnp.float32)
    zeros_deg = zeros_agg
    ones_deg = jnp.ones((_CHUNK, hid), jnp.float32)

    deg_part = degree(dst_p, ones_deg, zeros_deg)

    feat_pad = jnp.zeros((n_pad, in_dim), jnp.float32).at[:n].set(feat)
    dinv, hh, pooled0 = pl.pallas_call(
        functools.partial(_prep_body, n),
        out_shape=(
            jax.ShapeDtypeStruct((n_pad, 1), jnp.float32),
            jax.ShapeDtypeStruct((n_pad, hid), jnp.float32),
            jax.ShapeDtypeStruct((1, hid), jnp.float32),
        ),
    )(feat_pad, deg_part)

    layer_call = pl.pallas_call(
        functools.partial(_layer_body, n),
        out_shape=(
            jax.ShapeDtypeStruct((n_pad, hid), jnp.float32),
            jax.ShapeDtypeStruct((n_pad, hid), jnp.float32),
            jax.ShapeDtypeStruct((1, hid), jnp.float32),
        ),
    )

    h = feat_pad
    pooled = [pooled0]
    for i in range(num_layers):
        agg = spmm(hh, sd_p, zeros_agg)
        h, hh, p = layer_call(
            h, agg, dinv, cheb_W[i], cheb_b[i].reshape(1, hid),
            bn_gamma[i].reshape(1, hid), bn_beta[i].reshape(1, hid))
        pooled.append(p)

    p_all = jnp.concatenate(pooled, axis=0)
    log_probs, pooled_mean = pl.pallas_call(
        _head_body,
        out_shape=(
            jax.ShapeDtypeStruct((1, out_dim), jnp.float32),
            jax.ShapeDtypeStruct((1, hid), jnp.float32),
        ),
    )(p_all, lp_W, lp_b)
    return log_probs, pooled_mean


# speedup vs baseline: 1.1497x; 1.0019x over previous
import functools
import math

import jax
import jax.numpy as jnp
from jax import lax
from jax.experimental import pallas as pl
from jax.experimental.pallas import tpu as pltpu
from jax.experimental.pallas import tpu_sc as plsc

_NUM_CORES = 2
_NUM_TILES = 16
_CHUNK = 128


def _make_sc_kernels(n_pad, ch, hid):
    rows_per_tile = n_pad // _NUM_TILES
    ch_per_tile = ch // (_NUM_TILES * _NUM_CORES)
    cpt_pad = ch_per_tile + 8
    nbuf = 3
    mesh = plsc.VectorSubcoreMesh(core_axis_name="c", subcore_axis_name="s")

    nidx = nbuf + 1
    period = (nbuf * nidx) // math.gcd(nbuf, nidx)

    @functools.partial(
        pl.kernel,
        out_type=jax.ShapeDtypeStruct((_NUM_CORES, n_pad, hid), jnp.float32),
        mesh=mesh,
        scratch_types=(
            [pltpu.VMEM((nidx, 2, _CHUNK), jnp.int32),
             pltpu.VMEM((nbuf, _CHUNK, hid), jnp.float32),
             pltpu.VMEM_SHARED((n_pad, hid), jnp.float32)]
            + [pltpu.SemaphoreType.DMA] * (2 * nbuf + nidx)
        ),
    )
    def spmm(hh_hbm, sd_hbm, zeros_hbm, out_hbm, *rest):
        sdv3, rows3, aggs = rest[0], rest[1], rest[2]
        sdv = [sdv3.at[j] for j in range(nidx)]
        rows = [rows3.at[b] for b in range(nbuf)]
        sems = list(rest[3:3 + nbuf])
        ssems = list(rest[3 + nbuf:3 + 2 * nbuf])
        isems = list(rest[3 + 2 * nbuf:])
        cid = lax.axis_index("c")
        sid = lax.axis_index("s")
        sl = pl.ds(sid * rows_per_tile, rows_per_tile)
        base = (cid * _NUM_TILES + sid) * ch_per_tile
        pltpu.sync_copy(zeros_hbm, aggs.at[sl])
        plsc.subcore_barrier()

        def fire_idx(c, j):
            pltpu.async_copy(sd_hbm.at[base + c], sdv[j], isems[j])

        def fire_gather(j, b):
            pltpu.make_async_copy(sd_hbm.at[base], sdv[j], isems[j]).wait()
            pltpu.async_copy(hh_hbm.at[sdv[j].at[0]], rows[b], sems[b])

        def fire_scatter(j, b):
            pltpu.make_async_copy(hh_hbm.at[sdv[j].at[0]], rows[b],
                                  sems[b]).wait()
            pltpu.async_copy(rows[b], aggs.at[sdv[j].at[1]], ssems[b],
                             add=True)

        def wait_scatter(j, b):
            pltpu.make_async_copy(rows[b], aggs.at[sdv[j].at[1]],
                                  ssems[b]).wait()

        def drain_gather(j, b):
            pltpu.make_async_copy(hh_hbm.at[sdv[j].at[0]], rows[b],
                                  sems[b]).wait()

        for c in range(nbuf):
            fire_idx(c, c % nidx)
        for c in range(nbuf - 1):
            fire_gather(c % nidx, c % nbuf)

        def step(c, k, first=False):
            fire_scatter(k % nidx, k % nbuf)
            if not first:
                wait_scatter((k - 1) % nidx, (k + nbuf - 1) % nbuf)
            fire_gather((k + nbuf - 1) % nidx, (k + nbuf - 1) % nbuf)
            fire_idx(c + nbuf, (k + nbuf) % nidx)

        def body(tt, carry):
            for k in range(period):
                step(period * tt + k, k)
            return carry

        main = ch_per_tile // period
        for k in range(period):
            step(k, k, first=(k == 0))
        lax.fori_loop(1, main, body, 0)
        for i in range(ch_per_tile - main * period):
            c = main * period + i
            step(c, i)
        lk = ch_per_tile - 1
        wait_scatter(lk % nidx, lk % nbuf)
        for c in range(ch_per_tile, ch_per_tile + nbuf - 1):
            drain_gather(c % nidx, c % nbuf)
        j = (ch_per_tile + nbuf - 1) % nidx
        pltpu.make_async_copy(sd_hbm.at[base], sdv[j], isems[j]).wait()

        plsc.subcore_barrier()
        pltpu.sync_copy(aggs.at[sl], out_hbm.at[cid, sl])

    ndeg = 4

    @functools.partial(
        pl.kernel,
        out_type=jax.ShapeDtypeStruct((_NUM_CORES, n_pad, hid), jnp.float32),
        mesh=mesh,
        scratch_types=(
            [pltpu.VMEM((cpt_pad, _CHUNK), jnp.int32),
             pltpu.VMEM((_CHUNK, hid), jnp.float32),
             pltpu.VMEM_SHARED((n_pad, hid), jnp.float32)]
            + [pltpu.SemaphoreType.DMA] * ndeg
        ),
    )
    def degree(dst_hbm, ones_hbm, zeros_hbm, out_hbm, dsti, ones_v, degs,
               *dsems):
        cid = lax.axis_index("c")
        sid = lax.axis_index("s")
        sl = pl.ds(sid * rows_per_tile, rows_per_tile)
        base = (cid * _NUM_TILES + sid) * ch_per_tile
        pltpu.sync_copy(dst_hbm.at[pl.ds(base, cpt_pad)], dsti)
        pltpu.sync_copy(zeros_hbm, degs.at[sl])
        pltpu.sync_copy(ones_hbm, ones_v)
        plsc.subcore_barrier()

        def fire(t, k):
            pltpu.async_copy(ones_v, degs.at[dsti.at[t]], dsems[k],
                             add=True)

        def wait(t, k):
            pltpu.make_async_copy(ones_v, degs.at[dsti.at[t]],
                                  dsems[k]).wait()

        for k in range(ndeg):
            fire(k, k)

        def body(tt, carry):
            for k in range(ndeg):
                t = ndeg * tt + k
                wait(t - ndeg, k)
                fire(t, k)
            return carry

        lax.fori_loop(1, ch_per_tile // ndeg, body, 0)
        for k in range(ndeg):
            wait(ch_per_tile - ndeg + k, k)
        plsc.subcore_barrier()
        pltpu.sync_copy(degs.at[sl], out_hbm.at[cid, sl])

    return spmm, degree


def _prep_body(n, feat_ref, degp_ref, dinv_ref, hh_ref, pooled_ref):
    feat = feat_ref[...]
    deg = degp_ref[0, :, 0:1] + degp_ref[1, :, 0:1]
    dinv = lax.rsqrt(jnp.maximum(deg, 1.0))
    dinv_ref[...] = dinv
    hh_ref[...] = feat * dinv
    pooled_ref[...] = jnp.sum(feat, axis=0, keepdims=True)


def _layer_body(n, h_ref, agg_ref, dinv_ref, w_ref, b_ref, g_ref, beta_ref,
                hout_ref, hh_ref, pooled_ref):
    h = h_ref[...]
    n_pad, hid = h.shape
    dinv = dinv_ref[...]
    w = w_ref[...]
    x1 = -((agg_ref[0] + agg_ref[1]) * dinv)
    bf = jnp.bfloat16
    z = (jnp.dot(h.astype(bf), w[:hid].astype(bf),
                 preferred_element_type=jnp.float32)
         + jnp.dot(x1.astype(bf), w[hid:].astype(bf),
                   preferred_element_type=jnp.float32)
         + b_ref[...])
    rowmask = lax.broadcasted_iota(jnp.int32, (n_pad, 1), 0) < n
    zm = jnp.where(rowmask, z, 0.0)
    mean = jnp.sum(zm, axis=0, keepdims=True) / n
    d = jnp.where(rowmask, z - mean, 0.0)
    var = jnp.sum(d * d, axis=0, keepdims=True) / n
    hn = (z - mean) * lax.rsqrt(var + 1e-5) * g_ref[...] + beta_ref[...]
    hn = jnp.where(rowmask, jnp.maximum(hn, 0.0), 0.0)
    hout_ref[...] = hn
    pooled_ref[...] = jnp.sum(hn, axis=0, keepdims=True)
    hh_ref[...] = hn * dinv


def _head_body(p_ref, w_ref, b_ref, lp_ref, pm_ref):
    p = p_ref[...]
    w = w_ref[...]
    b = b_ref[...]
    reps = p.shape[0]
    s = jnp.zeros((1, w.shape[2]), jnp.float32)
    for i in range(reps):
        s = s + jnp.dot(p[i:i + 1], w[i], preferred_element_type=jnp.float32)
        s = s + b[i:i + 1]
    m = jnp.max(s)
    lse = m + jnp.log(jnp.sum(jnp.exp(s - m)))
    lp_ref[...] = s - lse
    pm_ref[...] = jnp.mean(p[1:], axis=0, keepdims=True)


def kernel(feat, edge_index, cheb_W, cheb_b, bn_gamma, bn_beta, lp_W, lp_b):
    n, in_dim = feat.shape
    e = edge_index.shape[1]
    num_layers, two_hid, hid = cheb_W.shape
    half = hid // 2
    out_dim = lp_W.shape[2]

    align = _NUM_TILES * 8
    n_pad = ((n + align - 1) // align) * align
    if n_pad == n:
        n_pad += align
    nch = (e + _CHUNK - 1) // _CHUNK
    grp = _NUM_TILES * _NUM_CORES * 8
    ch = ((nch + grp - 1) // grp) * grp
    pad_e = (ch + 8) * _CHUNK - e

    src = edge_index[0]
    dst = edge_index[1]
    lane = jnp.arange(pad_e, dtype=jnp.int32) % (n_pad - n)
    src_p = jnp.concatenate([src, n + lane]).reshape(ch + 8, _CHUNK)
    dst_p = jnp.concatenate([dst, n + lane]).reshape(ch + 8, _CHUNK)
    sd_p = jnp.stack([src_p, dst_p], axis=1)

    spmm, degree = _make_sc_kernels(n_pad, ch, hid)
    rows_per_tile = n_pad // _NUM_TILES
    zeros_agg = jnp.zeros((rows_per_tile, hid), jnp.float32)
    zeros_deg = zeros_agg
    ones_deg = jnp.ones((_CHUNK, hid), jnp.float32)

    deg_part = degree(dst_p, ones_deg, zeros_deg)

    feat_pad = jnp.zeros((n_pad, in_dim), jnp.float32).at[:n].set(feat)
    dinv, hh, pooled0 = pl.pallas_call(
        functools.partial(_prep_body, n),
        out_shape=(
            jax.ShapeDtypeStruct((n_pad, 1), jnp.float32),
            jax.ShapeDtypeStruct((n_pad, hid), jnp.float32),
            jax.ShapeDtypeStruct((1, hid), jnp.float32),
        ),
    )(feat_pad, deg_part)

    layer_call = pl.pallas_call(
        functools.partial(_layer_body, n),
        out_shape=(
            jax.ShapeDtypeStruct((n_pad, hid), jnp.float32),
            jax.ShapeDtypeStruct((n_pad, hid), jnp.float32),
            jax.ShapeDtypeStruct((1, hid), jnp.float32),
        ),
    )

    h = feat_pad
    pooled = [pooled0]
    for i in range(num_layers):
        agg = spmm(hh, sd_p, zeros_agg)
        h, hh, p = layer_call(
            h, agg, dinv, cheb_W[i], cheb_b[i].reshape(1, hid),
            bn_gamma[i].reshape(1, hid), bn_beta[i].reshape(1, hid))
        pooled.append(p)

    p_all = jnp.concatenate(pooled, axis=0)
    log_probs, pooled_mean = pl.pallas_call(
        _head_body,
        out_shape=(
            jax.ShapeDtypeStruct((1, out_dim), jnp.float32),
            jax.ShapeDtypeStruct((1, hid), jnp.float32),
        ),
    )(p_all, lp_W, lp_b)
    return log_probs, pooled_mean
